# baseline (speedup 1.0000x reference)
import functools

import jax
import jax.numpy as jnp
from jax import lax
from jax.experimental import pallas as pl
from jax.experimental.pallas import tpu as pltpu
from jax.experimental.pallas import tpu_sc as plsc

VOCAB = 1000000
EMBED_DIM = 32
BATCH = 4096
HIST_LEN = 200

N = BATCH * HIST_LEN
BLK = 128
NBLOCKS = N // BLK
NC, NS = 2, 16
NW = NC * NS
BPW = NBLOCKS // NW
NBUF = 5
GROUPS = BPW // NBUF


def _make_kernel():
    mesh = plsc.VectorSubcoreMesh(core_axis_name="c", subcore_axis_name="s")

    @functools.partial(
        pl.kernel,
        mesh=mesh,
        out_type=jax.ShapeDtypeStruct((N, EMBED_DIM), jnp.float32),
        compiler_params=pltpu.CompilerParams(use_tc_tiling_on_sc=False),
        scratch_types=[
            pltpu.VMEM((BPW, BLK), jnp.int32),
            pltpu.VMEM((NBUF, BLK, EMBED_DIM), jnp.float32),
            pltpu.SemaphoreType.DMA((NBUF,)),
            pltpu.SemaphoreType.DMA((NBUF,)),
        ],
    )
    def gather_kernel(table_hbm, idx_hbm, out_hbm, idx_v, rows_v, gsem, wsem):
        wid = lax.axis_index("s") * NC + lax.axis_index("c")
        blk_base = wid * BPW

        pltpu.sync_copy(idx_hbm.at[pl.ds(blk_base, BPW)], idx_v)

        for j in range(NBUF):
            pltpu.async_copy(table_hbm.at[idx_v.at[j]], rows_v.at[j], gsem.at[j])

        def group(g, _):
            for j in range(NBUF):
                blk = g * NBUF + j
                pltpu.make_async_copy(
                    table_hbm.at[idx_v.at[j]], rows_v.at[j], gsem.at[j]
                ).wait()
                pltpu.async_copy(
                    rows_v.at[j],
                    out_hbm.at[pl.ds((blk_base + blk) * BLK, BLK)],
                    wsem.at[j],
                )

            @pl.when(g + 1 < GROUPS)
            def _refire():
                for j in range(NBUF):
                    nxt = (g + 1) * NBUF + j
                    pltpu.make_async_copy(
                        rows_v.at[j],
                        out_hbm.at[pl.ds(blk_base * BLK, BLK)],
                        wsem.at[j],
                    ).wait()
                    pltpu.async_copy(
                        table_hbm.at[idx_v.at[nxt]], rows_v.at[j], gsem.at[j]
                    )

            return ()

        lax.fori_loop(0, GROUPS, group, ())

        for j in range(NBUF):
            pltpu.make_async_copy(
                rows_v.at[j],
                out_hbm.at[pl.ds(blk_base * BLK, BLK)],
                wsem.at[j],
            ).wait()

    return gather_kernel


_GATHER = _make_kernel()


@jax.jit
def kernel(indices, table):
    idx2d = indices.astype(jnp.int32).reshape(NBLOCKS, BLK)
    out = _GATHER(table, idx2d)
    return out.reshape(BATCH, HIST_LEN, EMBED_DIM)


# speedup vs baseline: 1.0049x; 1.0049x over previous
import functools

import jax
import jax.numpy as jnp
from jax import lax
from jax.experimental import pallas as pl
from jax.experimental.pallas import tpu as pltpu
from jax.experimental.pallas import tpu_sc as plsc

VOCAB = 1000000
EMBED_DIM = 32
BATCH = 4096
HIST_LEN = 200

N = BATCH * HIST_LEN
BLK = 128
NBLOCKS = N // BLK
NC, NS = 2, 16
NW = NC * NS
BPW = NBLOCKS // NW
NBUF = 8
GROUPS = BPW // NBUF


def _make_kernel():
    mesh = plsc.VectorSubcoreMesh(core_axis_name="c", subcore_axis_name="s")

    @functools.partial(
        pl.kernel,
        mesh=mesh,
        out_type=jax.ShapeDtypeStruct((N, EMBED_DIM), jnp.float32),
        compiler_params=pltpu.CompilerParams(use_tc_tiling_on_sc=False),
        scratch_types=[
            pltpu.VMEM((BPW, BLK), jnp.int32),
            pltpu.VMEM((NBUF, BLK, EMBED_DIM), jnp.float32),
            pltpu.SemaphoreType.DMA((NBUF,)),
            pltpu.SemaphoreType.DMA((NBUF,)),
        ],
    )
    def gather_kernel(table_hbm, idx_hbm, out_hbm, idx_v, rows_v, gsem, wsem):
        wid = lax.axis_index("s") * NC + lax.axis_index("c")
        blk_base = wid * BPW

        pltpu.sync_copy(idx_hbm.at[pl.ds(blk_base, BPW)], idx_v)

        for j in range(NBUF):
            pltpu.async_copy(table_hbm.at[idx_v.at[j]], rows_v.at[j], gsem.at[j])

        def group(g, _):
            for j in range(NBUF):
                blk = g * NBUF + j
                pltpu.make_async_copy(
                    table_hbm.at[idx_v.at[j]], rows_v.at[j], gsem.at[j]
                ).wait()
                pltpu.async_copy(
                    rows_v.at[j],
                    out_hbm.at[pl.ds((blk_base + blk) * BLK, BLK)],
                    wsem.at[j],
                )

            @pl.when(g + 1 < GROUPS)
            def _refire():
                for j in range(NBUF):
                    nxt = (g + 1) * NBUF + j
                    pltpu.make_async_copy(
                        rows_v.at[j],
                        out_hbm.at[pl.ds(blk_base * BLK, BLK)],
                        wsem.at[j],
                    ).wait()
                    pltpu.async_copy(
                        table_hbm.at[idx_v.at[nxt]], rows_v.at[j], gsem.at[j]
                    )

            return ()

        lax.fori_loop(0, GROUPS, group, ())

        for j in range(NBUF):
            pltpu.make_async_copy(
                rows_v.at[j],
                out_hbm.at[pl.ds(blk_base * BLK, BLK)],
                wsem.at[j],
            ).wait()

    return gather_kernel


_GATHER = _make_kernel()


@jax.jit
def kernel(indices, table):
    idx2d = indices.astype(jnp.int32).reshape(NBLOCKS, BLK)
    out = _GATHER(table, idx2d)
    return out.reshape(BATCH, HIST_LEN, EMBED_DIM)
